# baseline (speedup 1.0000x reference)
import functools

import jax
import jax.numpy as jnp
from jax import lax
from jax.experimental import pallas as pl
from jax.experimental.pallas import tpu as pltpu
from jax.experimental.pallas import tpu_sc as plsc

T = 2048
D = 1024
H = 1024
NE = 8
K = 2
PAIRS = T * K
TILE = 256
NSLOTS = 6144
NTILES = NSLOTS // TILE


NCHUNK = 32
TOK_PER_CHUNK = T // NCHUNK


def _router_body(x_ref, wg_ref, tri_ref, ids_ref, sc_ref, cnt_ref, aux_ref):
    x = x_ref[...]
    wg = wg_ref[...]
    logits = jnp.dot(x, wg, preferred_element_type=jnp.float32)
    lane = lax.broadcasted_iota(jnp.int32, logits.shape, 1)
    neg = jnp.float32(-1e30)
    logits = jnp.where(lane < NE, logits, neg)
    m1 = jnp.max(logits, axis=1, keepdims=True)
    i1 = jnp.min(jnp.where(logits == m1, lane, 128), axis=1, keepdims=True)
    l2 = jnp.where(lane == i1, neg, logits)
    m2 = jnp.max(l2, axis=1, keepdims=True)
    i2 = jnp.min(jnp.where(l2 == m2, lane, 128), axis=1, keepdims=True)
    z = jnp.sum(jnp.exp(logits - m1), axis=1, keepdims=True)
    s1 = 1.0 / z
    s2 = jnp.exp(m2 - m1) / z
    ids_ref[...] = jnp.concatenate([i1, i2], axis=1)
    sc_ref[...] = jnp.concatenate([s1, s2], axis=1)
    lane3 = lax.broadcasted_iota(jnp.int32, (NCHUNK, TOK_PER_CHUNK, 128), 2)
    i1r = i1.reshape(NCHUNK, TOK_PER_CHUNK, 1)
    i2r = i2.reshape(NCHUNK, TOK_PER_CHUNK, 1)
    hits = (lane3 == i1r).astype(jnp.int32) + (lane3 == i2r).astype(jnp.int32)
    cnts = jnp.sum(hits, axis=1)
    cnt_ref[...] = cnts
    totals = jnp.sum(cnts, axis=0, keepdims=True).astype(jnp.float32)
    padded = jnp.floor((totals + (TILE - 1)) / TILE) * TILE
    offs = jnp.dot(padded, tri_ref[...],
                   preferred_element_type=jnp.float32)
    offs_i = offs.astype(jnp.int32)
    lane2 = lax.broadcasted_iota(jnp.int32, (1, 128), 1)
    te = jnp.zeros((1, 128), jnp.int32) - 1
    for e in range(NE):
        tstart_e = offs_i[0, e] // TILE
        te = te + (lane2 >= tstart_e).astype(jnp.int32)
    aux_ref[...] = jnp.concatenate([offs_i, te], axis=0)


def _router(x_flat, wg_pad, tri):
    return pl.pallas_call(
        _router_body,
        out_shape=(
            jax.ShapeDtypeStruct((T, K), jnp.int32),
            jax.ShapeDtypeStruct((T, K), jnp.float32),
            jax.ShapeDtypeStruct((NCHUNK, 128), jnp.int32),
            jax.ShapeDtypeStruct((2, 128), jnp.int32),
        ),
    )(x_flat, wg_pad, tri)


@functools.cache
def _dispatch_kernel_build():
    mesh = plsc.VectorSubcoreMesh(core_axis_name="c", subcore_axis_name="s")
    return pl.kernel(
        _dispatch_body,
        out_type=(
            jax.ShapeDtypeStruct((PAIRS,), jnp.int32),
            jax.ShapeDtypeStruct((NSLOTS, D), jnp.float32),
        ),
        mesh=mesh,
        scratch_types=[
            pltpu.VMEM((NCHUNK, 16), jnp.int32),
            pltpu.VMEM((16,), jnp.int32),
            pltpu.VMEM((128,), jnp.int32),
            pltpu.VMEM((4, 32), jnp.int32),
            pltpu.VMEM((4, 32), jnp.int32),
            pltpu.VMEM((2, 32, D), jnp.float32),
            pltpu.SemaphoreType.DMA,
            pltpu.SemaphoreType.DMA,
        ],
        compiler_params=pltpu.CompilerParams(needs_layout_passes=False),
    )


def _dispatch_body(ids_hbm, cnts_hbm, offs_hbm, x_hbm, pos_hbm, xg_hbm,
                   cbuf, offbuf, idv, posbuf, tokidx, rows, gsem, ssem):
    w = lax.axis_index("s") * 2 + lax.axis_index("c")
    pltpu.sync_copy(cnts_hbm, cbuf)
    pltpu.sync_copy(offs_hbm, offbuf)
    pltpu.sync_copy(ids_hbm.at[pl.ds(w * 128, 128)], idv)
    lane = lax.iota(jnp.int32, 16)
    zero = jnp.zeros((16,), jnp.int32)
    basev = offbuf[...]
    for t in range(NCHUNK):
        pred = jnp.where(t < w, 1, 0).astype(jnp.int32)
        basev = basev + cbuf[t, :] * pred
    for s4 in range(4):
        for h in range(2):
            vidx = s4 * 32 + h * 16
            v = idv[pl.ds(vidx, 16)]
            pos_v = zero
            hist = zero
            for e in range(NE):
                m = v == e
                inc = plsc.cumsum(jnp.where(m, 1, 0).astype(jnp.int32))
                pos_v = jnp.where(m, basev[e] + inc - 1, pos_v)
                pc = plsc.all_reduce_population_count(m)
                hist = jnp.where(lane == e, pc, hist)
            basev = basev + hist
            posbuf[s4, pl.ds(h * 16, 16)] = pos_v
            tokidx[s4, pl.ds(h * 16, 16)] = (w * 128 + vidx + lane) // 2
        pltpu.sync_copy(posbuf.at[s4],
                        pos_hbm.at[pl.ds(w * 128 + s4 * 32, 32)])
    g = [None] * 4
    sc = [None] * 4
    g[0] = pltpu.async_copy(x_hbm.at[tokidx.at[0]], rows.at[0], gsem)
    for s4 in range(4):
        if s4 >= 1:
            sc[s4 - 1].wait()
        if s4 + 1 < 4:
            g[s4 + 1] = pltpu.async_copy(x_hbm.at[tokidx.at[s4 + 1]],
                                         rows.at[(s4 + 1) % 2], gsem)
        g[s4].wait()
        sc[s4] = pltpu.async_copy(rows.at[s4 % 2], xg_hbm.at[posbuf.at[s4]],
                                  ssem)
    sc[3].wait()


@functools.cache
def _combine_kernel_build():
    mesh = plsc.VectorSubcoreMesh(core_axis_name="c", subcore_axis_name="s")
    return pl.kernel(
        _combine_body,
        out_type=jax.ShapeDtypeStruct((T, D), jnp.float32),
        mesh=mesh,
        scratch_types=[
            pltpu.VMEM((4, 32), jnp.int32),
            pltpu.VMEM((32,), jnp.float32),
            pltpu.VMEM((32, D), jnp.float32),
            pltpu.VMEM((16, D), jnp.float32),
            pltpu.SemaphoreType.DMA,
        ],
        compiler_params=pltpu.CompilerParams(needs_layout_passes=False),
    )


def _combine_body(y_hbm, pos_hbm, sc_hbm, out_hbm,
                  posbuf, sbuf, yrows, obuf, sem):
    w = lax.axis_index("s") * 2 + lax.axis_index("c")
    for s4 in range(4):
        pltpu.sync_copy(pos_hbm.at[pl.ds(w * 128 + s4 * 32, 32)],
                        posbuf.at[s4])
        pltpu.sync_copy(sc_hbm.at[pl.ds(w * 128 + s4 * 32, 32)], sbuf)
        pltpu.async_copy(y_hbm.at[posbuf.at[s4]], yrows, sem).wait()
        sv0 = sbuf[pl.ds(0, 16)]
        sv1 = sbuf[pl.ds(16, 16)]
        for i in range(16):
            sv = sv0 if 2 * i < 16 else sv1
            sa = sv[(2 * i) % 16]
            sb = sv[(2 * i + 1) % 16]

            def body(vi, _, i=i, sa=sa, sb=sb):
                for u in range(4):
                    c = vi * 64 + u * 16
                    a = yrows[2 * i, pl.ds(c, 16)]
                    bv = yrows[2 * i + 1, pl.ds(c, 16)]
                    obuf[i, pl.ds(c, 16)] = sa * a + sb * bv
                return 0

            lax.fori_loop(0, D // 64, body, 0)
        pltpu.sync_copy(obuf, out_hbm.at[pl.ds(w * 64 + s4 * 16, 16)])


def _expert_body(eid_ref, xg_ref, w1_ref, w2_ref, wc_ref, y_ref):
    xg = xg_ref[...]
    w1 = w1_ref[0]
    w2 = w2_ref[0]
    wc = wc_ref[0]
    dn = (((1,), (1,)), ((), ()))
    h1 = lax.dot_general(xg, w1, dn, preferred_element_type=jnp.float32)
    h2 = lax.dot_general(xg, w2, dn, preferred_element_type=jnp.float32)
    h = (h1 * jax.nn.sigmoid(h1)) * h2
    y_ref[...] = lax.dot_general(h, wc, dn, preferred_element_type=jnp.float32)


def _expert_mlp(xg, w1, w2, wc, tile_eid):
    grid_spec = pltpu.PrefetchScalarGridSpec(
        num_scalar_prefetch=1,
        grid=(NTILES,),
        in_specs=[
            pl.BlockSpec((TILE, D), lambda i, eid: (i, 0)),
            pl.BlockSpec((1, H, D), lambda i, eid: (eid[i], 0, 0)),
            pl.BlockSpec((1, H, D), lambda i, eid: (eid[i], 0, 0)),
            pl.BlockSpec((1, D, H), lambda i, eid: (eid[i], 0, 0)),
        ],
        out_specs=pl.BlockSpec((TILE, D), lambda i, eid: (i, 0)),
    )
    return pl.pallas_call(
        _expert_body,
        grid_spec=grid_spec,
        out_shape=jax.ShapeDtypeStruct((NSLOTS, D), jnp.float32),
        compiler_params=pltpu.CompilerParams(
            dimension_semantics=("arbitrary",),
        ),
    )(tile_eid, xg, w1, w2, wc)


def kernel(x, W1, W2, Wc, Wg):
    b, s, d = x.shape
    x_flat = x.reshape(T, D)
    wg_pad = jnp.zeros((D, 128), jnp.float32).at[:, :NE].set(Wg.T)
    tri = (jnp.arange(128)[:, None] < jnp.arange(128)[None, :]
           ).astype(jnp.float32)
    ids, scores, cnts, aux = _router(x_flat, wg_pad, tri)
    tile_eid = aux[1, :NTILES]
    pos, xg = _dispatch_kernel_build()(ids.reshape(-1), cnts[:, :16],
                                       aux[0, :16], x_flat)
    y = _expert_mlp(xg, W1, W2, Wc, tile_eid)
    out = _combine_kernel_build()(y, pos, scores.reshape(-1))
    return out.reshape(b, s, d)


# speedup vs baseline: 1.2134x; 1.2134x over previous
import functools

import jax
import jax.numpy as jnp
from jax import lax
from jax.experimental import pallas as pl
from jax.experimental.pallas import tpu as pltpu
from jax.experimental.pallas import tpu_sc as plsc

T = 2048
D = 1024
H = 1024
NE = 8
K = 2
PAIRS = T * K
TILE = 256
NSLOTS = 6144
NTILES = NSLOTS // TILE


NCHUNK = 32
TOK_PER_CHUNK = T // NCHUNK


def _router_body(x_ref, wg_ref, tri_ref, ids_ref, sc_ref, cnt_ref, offs_ref,
                 te_ref):
    x = x_ref[...]
    wg = wg_ref[...]
    logits = jnp.dot(x, wg, preferred_element_type=jnp.float32)
    lane = lax.broadcasted_iota(jnp.int32, logits.shape, 1)
    neg = jnp.float32(-1e30)
    logits = jnp.where(lane < NE, logits, neg)
    m1 = jnp.max(logits, axis=1, keepdims=True)
    i1 = jnp.min(jnp.where(logits == m1, lane, 128), axis=1, keepdims=True)
    l2 = jnp.where(lane == i1, neg, logits)
    m2 = jnp.max(l2, axis=1, keepdims=True)
    i2 = jnp.min(jnp.where(l2 == m2, lane, 128), axis=1, keepdims=True)
    z = jnp.sum(jnp.exp(logits - m1), axis=1, keepdims=True)
    s1 = 1.0 / z
    s2 = jnp.exp(m2 - m1) / z
    ids_ref[...] = jnp.concatenate([i1, i2], axis=1)
    sc_ref[...] = jnp.concatenate([s1, s2], axis=1)
    nck = NCHUNK // K
    tpc = T // nck
    lane3 = lax.broadcasted_iota(jnp.int32, (nck, tpc, 128), 2)
    h1 = jnp.sum((lane3 == i1.reshape(nck, tpc, 1)).astype(jnp.int32), axis=1)
    h2 = jnp.sum((lane3 == i2.reshape(nck, tpc, 1)).astype(jnp.int32), axis=1)
    cnts = jnp.concatenate([h1, h2], axis=0)
    cnt_ref[...] = cnts[:, :16]
    totals = jnp.sum(cnts, axis=0, keepdims=True).astype(jnp.float32)
    padded = jnp.floor((totals + (TILE - 1)) / TILE) * TILE
    offs = jnp.dot(padded, tri_ref[...],
                   preferred_element_type=jnp.float32)
    offs_i = offs.astype(jnp.int32)
    offs_ref[...] = offs_i[:, :16]
    lane2 = lax.broadcasted_iota(jnp.int32, (1, 128), 1)
    te = jnp.zeros((1, 128), jnp.int32) - 1
    for e in range(NE):
        tstart_e = offs_i[0, e] // TILE
        te = te + (lane2 >= tstart_e).astype(jnp.int32)
    te_ref[...] = te


def _router(x_flat, wg_pad, tri):
    return pl.pallas_call(
        _router_body,
        out_shape=(
            jax.ShapeDtypeStruct((T, K), jnp.int32),
            jax.ShapeDtypeStruct((T, K), jnp.float32),
            jax.ShapeDtypeStruct((NCHUNK, 16), jnp.int32),
            jax.ShapeDtypeStruct((1, 16), jnp.int32),
            jax.ShapeDtypeStruct((1, 128), jnp.int32),
        ),
    )(x_flat, wg_pad, tri)


@functools.cache
def _dispatch_kernel_build():
    mesh = plsc.VectorSubcoreMesh(core_axis_name="c", subcore_axis_name="s", num_cores=2, num_subcores=16)
    return pl.kernel(
        _dispatch_body,
        out_type=(
            jax.ShapeDtypeStruct((PAIRS,), jnp.int32),
            jax.ShapeDtypeStruct((NSLOTS, D), jnp.float32),
            jax.ShapeDtypeStruct((NSLOTS, 128), jnp.float32),
        ),
        mesh=mesh,
        scratch_types=[
            pltpu.VMEM((NCHUNK, 16), jnp.int32),
            pltpu.VMEM((1, 16), jnp.int32),
            pltpu.VMEM((128,), jnp.int32),
            pltpu.VMEM((128,), jnp.float32),
            pltpu.VMEM((4, 32), jnp.int32),
            pltpu.VMEM((4, 32, 128), jnp.float32),
            pltpu.VMEM((2, 32, D), jnp.float32),
            pltpu.SemaphoreType.DMA,
            pltpu.SemaphoreType.DMA,
        ],
        compiler_params=pltpu.CompilerParams(needs_layout_passes=False),
    )


def _dispatch_body(ids_hbm, cnts_hbm, offs_hbm, sc_hbm, x_hbm,
                   pos_hbm, xg_hbm, slotw_hbm,
                   cbuf, offbuf, idv, scv, posbuf, swbuf, rows, gsem, ssem):
    w = lax.axis_index("s") * 2 + lax.axis_index("c")
    tokbase = (w % (NCHUNK // K)) * 128
    pltpu.sync_copy(cnts_hbm, cbuf)
    pltpu.sync_copy(offs_hbm, offbuf)
    pltpu.sync_copy(ids_hbm.at[pl.ds(w * 128, 128)], idv)
    pltpu.sync_copy(sc_hbm.at[pl.ds(w * 128, 128)], scv)
    lane = lax.iota(jnp.int32, 16)
    zero = jnp.zeros((16,), jnp.int32)
    basev = offbuf[0, :]
    for t in range(NCHUNK):
        pred = jnp.where(t < w, 1, 0).astype(jnp.int32)
        basev = basev + cbuf[t, :] * pred
    for s4 in range(4):
        for h in range(2):
            vidx = s4 * 32 + h * 16
            v = idv[pl.ds(vidx, 16)]
            pos_v = zero
            hist = zero
            for e in range(NE):
                m = v == e
                inc = plsc.cumsum(jnp.where(m, 1, 0).astype(jnp.int32))
                pos_v = jnp.where(m, basev[e] + inc - 1, pos_v)
                pc = plsc.all_reduce_population_count(m)
                hist = jnp.where(lane == e, pc, hist)
            basev = basev + hist
            posbuf[s4, pl.ds(h * 16, 16)] = pos_v
            sv = scv[pl.ds(vidx, 16)]
            for j in range(16):
                swbuf[s4, h * 16 + j, pl.ds(0, 16)] = (
                    jnp.zeros((16,), jnp.float32) + sv[j])
        pltpu.sync_copy(posbuf.at[s4],
                        pos_hbm.at[pl.ds(w * 128 + s4 * 32, 32)])
    g = [None] * 4
    sc = [None] * 4
    g[0] = pltpu.async_copy(x_hbm.at[pl.ds(tokbase, 32)], rows.at[0], gsem)
    for s4 in range(4):
        if s4 >= 1:
            sc[s4 - 1].wait()
        if s4 + 1 < 4:
            g[s4 + 1] = pltpu.async_copy(
                x_hbm.at[pl.ds(tokbase + (s4 + 1) * 32, 32)],
                rows.at[(s4 + 1) % 2], gsem)
        g[s4].wait()
        sc[s4] = pltpu.async_copy(rows.at[s4 % 2], xg_hbm.at[posbuf.at[s4]],
                                  ssem)
        pltpu.async_copy(swbuf.at[s4], slotw_hbm.at[posbuf.at[s4]],
                         ssem).wait()
    sc[3].wait()


@functools.cache
def _combine_kernel_build():
    mesh = plsc.VectorSubcoreMesh(core_axis_name="c", subcore_axis_name="s", num_cores=2, num_subcores=16)
    return pl.kernel(
        _combine_body,
        out_type=jax.ShapeDtypeStruct((T, D), jnp.float32),
        mesh=mesh,
        scratch_types=[
            pltpu.VMEM((8, 16), jnp.int32),
            pltpu.VMEM((2, 16, D), jnp.float32),
            pltpu.VMEM((2, 16, D), jnp.float32),
            pltpu.SemaphoreType.DMA,
            pltpu.SemaphoreType.DMA,
        ],
        compiler_params=pltpu.CompilerParams(needs_layout_passes=False),
    )


def _combine_body(y_hbm, pos_hbm, out_hbm, posbuf, abuf, bbuf, gsem, osem):
    w = lax.axis_index("s") * 2 + lax.axis_index("c")
    for s4 in range(4):
        pltpu.sync_copy(pos_hbm.at[pl.ds(w * 64 + s4 * 16, 16)],
                        posbuf.at[s4])
        pltpu.sync_copy(pos_hbm.at[pl.ds(T + w * 64 + s4 * 16, 16)],
                        posbuf.at[4 + s4])

    def gather(s4):
        bi = s4 % 2
        ga = pltpu.async_copy(y_hbm.at[posbuf.at[s4]], abuf.at[bi], gsem)
        gb = pltpu.async_copy(y_hbm.at[posbuf.at[4 + s4]], bbuf.at[bi], gsem)
        return ga, gb

    g = [None] * 4
    osc = [None] * 4
    g[0] = gather(0)
    for s4 in range(4):
        bi = s4 % 2
        if s4 + 1 < 4:
            if s4 >= 1:
                osc[s4 - 1].wait()
            g[s4 + 1] = gather(s4 + 1)
        g[s4][0].wait()
        g[s4][1].wait()

        @plsc.parallel_loop(0, 16 * (D // 16), 1, unroll=4)
        def _add(i, bi=bi):
            r = lax.shift_right_logical(i, 6)
            c = lax.shift_left(jnp.bitwise_and(i, D // 16 - 1), 4)
            a = abuf[bi, r, pl.ds(c, 16)]
            bv = bbuf[bi, r, pl.ds(c, 16)]
            abuf[bi, r, pl.ds(c, 16)] = a + bv

        osc[s4] = pltpu.async_copy(
            abuf.at[bi], out_hbm.at[pl.ds(w * 64 + s4 * 16, 16)], osem)
    osc[2].wait()
    osc[3].wait()


def _expert_body(eid_ref, xg_ref, w1_ref, w2_ref, wc_ref, sw_ref, y_ref):
    xg = xg_ref[...]
    w1 = w1_ref[0]
    w2 = w2_ref[0]
    wc = wc_ref[0]
    dn = (((1,), (1,)), ((), ()))
    h1 = lax.dot_general(xg, w1, dn, preferred_element_type=jnp.float32)
    h2 = lax.dot_general(xg, w2, dn, preferred_element_type=jnp.float32)
    h = (h1 * jax.nn.sigmoid(h1)) * h2
    eo = lax.dot_general(h, wc, dn, preferred_element_type=jnp.float32)
    y_ref[...] = eo * sw_ref[:, 0:1]


def _expert_mlp(xg, w1, w2, wc, slotw, tile_eid):
    grid_spec = pltpu.PrefetchScalarGridSpec(
        num_scalar_prefetch=1,
        grid=(NTILES,),
        in_specs=[
            pl.BlockSpec((TILE, D), lambda i, eid: (i, 0)),
            pl.BlockSpec((1, H, D), lambda i, eid: (eid[0, i], 0, 0)),
            pl.BlockSpec((1, H, D), lambda i, eid: (eid[0, i], 0, 0)),
            pl.BlockSpec((1, D, H), lambda i, eid: (eid[0, i], 0, 0)),
            pl.BlockSpec((TILE, 128), lambda i, eid: (i, 0)),
        ],
        out_specs=pl.BlockSpec((TILE, D), lambda i, eid: (i, 0)),
    )
    return pl.pallas_call(
        _expert_body,
        grid_spec=grid_spec,
        out_shape=jax.ShapeDtypeStruct((NSLOTS, D), jnp.float32),
        compiler_params=pltpu.CompilerParams(
            dimension_semantics=("arbitrary",),
        ),
    )(tile_eid, xg, w1, w2, wc, slotw)


def kernel(x, W1, W2, Wc, Wg):
    b, s, d = x.shape
    x_flat = x.reshape(T, D)
    wg_pad = jnp.zeros((D, 128), jnp.float32).at[:, :NE].set(Wg.T)
    tri = (jnp.arange(128)[:, None] < jnp.arange(128)[None, :]
           ).astype(jnp.float32)
    ids, scores, cnts, offs, te = _router(x_flat, wg_pad, tri)
    pos, xg, slotw = _dispatch_kernel_build()(ids.T.reshape(-1), cnts, offs,
                                              scores.T.reshape(-1), x_flat)
    y = _expert_mlp(xg, W1, W2, Wc, slotw, te)
    out = _combine_kernel_build()(y, pos)
    return out.reshape(b, s, d)


# speedup vs baseline: 1.2138x; 1.0003x over previous
import functools

import jax
import jax.numpy as jnp
from jax import lax
from jax.experimental import pallas as pl
from jax.experimental.pallas import tpu as pltpu
from jax.experimental.pallas import tpu_sc as plsc

T = 2048
D = 1024
H = 1024
NE = 8
K = 2
PAIRS = T * K
TILE = 256
NSLOTS = 6144
NTILES = NSLOTS // TILE


NCHUNK = 32
TOK_PER_CHUNK = T // NCHUNK


def _router_body(x_ref, wg_ref, tri_ref, ids_ref, sc_ref, cnt_ref, offs_ref,
                 te_ref):
    x = x_ref[...]
    wg = wg_ref[...]
    logits = jnp.dot(x, wg, preferred_element_type=jnp.float32)
    lane = lax.broadcasted_iota(jnp.int32, logits.shape, 1)
    neg = jnp.float32(-1e30)
    logits = jnp.where(lane < NE, logits, neg)
    m1 = jnp.max(logits, axis=1, keepdims=True)
    i1 = jnp.min(jnp.where(logits == m1, lane, 128), axis=1, keepdims=True)
    l2 = jnp.where(lane == i1, neg, logits)
    m2 = jnp.max(l2, axis=1, keepdims=True)
    i2 = jnp.min(jnp.where(l2 == m2, lane, 128), axis=1, keepdims=True)
    z = jnp.sum(jnp.exp(logits - m1), axis=1, keepdims=True)
    s1 = 1.0 / z
    s2 = jnp.exp(m2 - m1) / z
    ids_ref[...] = jnp.concatenate([i1, i2], axis=1)
    sc_ref[...] = jnp.concatenate([s1, s2], axis=1)
    nck = NCHUNK // K
    tpc = T // nck
    lane3 = lax.broadcasted_iota(jnp.int32, (nck, tpc, 128), 2)
    h1 = jnp.sum((lane3 == i1.reshape(nck, tpc, 1)).astype(jnp.int32), axis=1)
    h2 = jnp.sum((lane3 == i2.reshape(nck, tpc, 1)).astype(jnp.int32), axis=1)
    cnts = jnp.concatenate([h1, h2], axis=0)
    cnt_ref[...] = cnts[:, :16]
    totals = jnp.sum(cnts, axis=0, keepdims=True).astype(jnp.float32)
    padded = jnp.floor((totals + (TILE - 1)) / TILE) * TILE
    offs = jnp.dot(padded, tri_ref[...],
                   preferred_element_type=jnp.float32)
    offs_i = offs.astype(jnp.int32)
    offs_ref[...] = offs_i[:, :16]
    lane2 = lax.broadcasted_iota(jnp.int32, (1, 128), 1)
    te = jnp.zeros((1, 128), jnp.int32) - 1
    for e in range(NE):
        tstart_e = offs_i[0, e] // TILE
        te = te + (lane2 >= tstart_e).astype(jnp.int32)
    te_ref[...] = te


def _router(x_flat, wg_pad, tri):
    return pl.pallas_call(
        _router_body,
        out_shape=(
            jax.ShapeDtypeStruct((T, K), jnp.int32),
            jax.ShapeDtypeStruct((T, K), jnp.float32),
            jax.ShapeDtypeStruct((NCHUNK, 16), jnp.int32),
            jax.ShapeDtypeStruct((1, 16), jnp.int32),
            jax.ShapeDtypeStruct((1, 128), jnp.int32),
        ),
    )(x_flat, wg_pad, tri)


@functools.cache
def _dispatch_kernel_build():
    mesh = plsc.VectorSubcoreMesh(core_axis_name="c", subcore_axis_name="s", num_cores=2, num_subcores=16)
    return pl.kernel(
        _dispatch_body,
        out_type=(
            jax.ShapeDtypeStruct((PAIRS,), jnp.int32),
            jax.ShapeDtypeStruct((NSLOTS, D), jnp.float32),
            jax.ShapeDtypeStruct((NSLOTS, 128), jnp.float32),
        ),
        mesh=mesh,
        scratch_types=[
            pltpu.VMEM((NCHUNK, 16), jnp.int32),
            pltpu.VMEM((1, 16), jnp.int32),
            pltpu.VMEM((128,), jnp.int32),
            pltpu.VMEM((128,), jnp.float32),
            pltpu.VMEM((4, 32), jnp.int32),
            pltpu.VMEM((4, 32, 128), jnp.float32),
            pltpu.VMEM((2, 32, D), jnp.float32),
            pltpu.SemaphoreType.DMA,
            pltpu.SemaphoreType.DMA,
        ],
        compiler_params=pltpu.CompilerParams(needs_layout_passes=False),
    )


def _dispatch_body(ids_hbm, cnts_hbm, offs_hbm, sc_hbm, x_hbm,
                   pos_hbm, xg_hbm, slotw_hbm,
                   cbuf, offbuf, idv, scv, posbuf, swbuf, rows, gsem, ssem):
    w = lax.axis_index("s") * 2 + lax.axis_index("c")
    tokbase = (w % (NCHUNK // K)) * 128
    pltpu.sync_copy(cnts_hbm, cbuf)
    pltpu.sync_copy(offs_hbm, offbuf)
    pltpu.sync_copy(ids_hbm.at[pl.ds(w * 128, 128)], idv)
    pltpu.sync_copy(sc_hbm.at[pl.ds(w * 128, 128)], scv)
    lane = lax.iota(jnp.int32, 16)
    zero = jnp.zeros((16,), jnp.int32)
    basev = offbuf[0, :]
    for t in range(NCHUNK):
        pred = jnp.where(t < w, 1, 0).astype(jnp.int32)
        basev = basev + cbuf[t, :] * pred
    for s4 in range(4):
        for h in range(2):
            vidx = s4 * 32 + h * 16
            v = idv[pl.ds(vidx, 16)]
            pos_v = zero
            hist = zero
            for e in range(NE):
                m = v == e
                inc = plsc.cumsum(jnp.where(m, 1, 0).astype(jnp.int32))
                pos_v = jnp.where(m, basev[e] + inc - 1, pos_v)
                pc = plsc.all_reduce_population_count(m)
                hist = jnp.where(lane == e, pc, hist)
            basev = basev + hist
            posbuf[s4, pl.ds(h * 16, 16)] = pos_v
            sv = scv[pl.ds(vidx, 16)]
            for j in range(16):
                swbuf[s4, h * 16 + j, pl.ds(0, 16)] = (
                    jnp.zeros((16,), jnp.float32) + sv[j])
        pltpu.sync_copy(posbuf.at[s4],
                        pos_hbm.at[pl.ds(w * 128 + s4 * 32, 32)])
    g = [None] * 4
    sc = [None] * 4
    g[0] = pltpu.async_copy(x_hbm.at[pl.ds(tokbase, 32)], rows.at[0], gsem)
    for s4 in range(4):
        if s4 >= 1:
            sc[s4 - 1].wait()
        if s4 + 1 < 4:
            g[s4 + 1] = pltpu.async_copy(
                x_hbm.at[pl.ds(tokbase + (s4 + 1) * 32, 32)],
                rows.at[(s4 + 1) % 2], gsem)
        g[s4].wait()
        sc[s4] = pltpu.async_copy(rows.at[s4 % 2], xg_hbm.at[posbuf.at[s4]],
                                  ssem)
        pltpu.async_copy(swbuf.at[s4], slotw_hbm.at[posbuf.at[s4]],
                         ssem).wait()
    sc[3].wait()


@functools.cache
def _combine_kernel_build():
    mesh = plsc.VectorSubcoreMesh(core_axis_name="c", subcore_axis_name="s", num_cores=2, num_subcores=16)
    return pl.kernel(
        _combine_body,
        out_type=jax.ShapeDtypeStruct((T, D), jnp.float32),
        mesh=mesh,
        scratch_types=[
            pltpu.VMEM((8, 16), jnp.int32),
            pltpu.VMEM((2, 16, D), jnp.float32),
            pltpu.VMEM((2, 16, D), jnp.float32),
            pltpu.SemaphoreType.DMA,
            pltpu.SemaphoreType.DMA,
        ],
        compiler_params=pltpu.CompilerParams(needs_layout_passes=False),
    )


def _combine_body(y_hbm, pos_hbm, out_hbm, posbuf, abuf, bbuf, gsem, osem):
    w = lax.axis_index("s") * 2 + lax.axis_index("c")
    for s4 in range(4):
        pltpu.sync_copy(pos_hbm.at[pl.ds(w * 64 + s4 * 16, 16)],
                        posbuf.at[s4])
        pltpu.sync_copy(pos_hbm.at[pl.ds(T + w * 64 + s4 * 16, 16)],
                        posbuf.at[4 + s4])

    def gather(s4):
        bi = s4 % 2
        ga = pltpu.async_copy(y_hbm.at[posbuf.at[s4]], abuf.at[bi], gsem)
        gb = pltpu.async_copy(y_hbm.at[posbuf.at[4 + s4]], bbuf.at[bi], gsem)
        return ga, gb

    g = [None] * 4
    osc = [None] * 4
    g[0] = gather(0)
    for s4 in range(4):
        bi = s4 % 2
        if s4 + 1 < 4:
            if s4 >= 1:
                osc[s4 - 1].wait()
            g[s4 + 1] = gather(s4 + 1)
        g[s4][0].wait()
        g[s4][1].wait()

        @plsc.parallel_loop(0, 16 * (D // 16), 1, unroll=4)
        def _add(i, bi=bi):
            r = lax.shift_right_logical(i, 6)
            c = lax.shift_left(jnp.bitwise_and(i, D // 16 - 1), 4)
            a = abuf[bi, r, pl.ds(c, 16)]
            bv = bbuf[bi, r, pl.ds(c, 16)]
            abuf[bi, r, pl.ds(c, 16)] = a + bv

        osc[s4] = pltpu.async_copy(
            abuf.at[bi], out_hbm.at[pl.ds(w * 64 + s4 * 16, 16)], osem)
    osc[2].wait()
    osc[3].wait()


def _expert_body(eid_ref, xg_ref, w1_ref, w2_ref, wc_ref, sw_ref, y_ref):
    xg = xg_ref[...].astype(jnp.bfloat16)
    w1 = w1_ref[0].astype(jnp.bfloat16)
    w2 = w2_ref[0].astype(jnp.bfloat16)
    wc = wc_ref[0].astype(jnp.bfloat16)
    dn = (((1,), (1,)), ((), ()))
    h1 = lax.dot_general(xg, w1, dn, preferred_element_type=jnp.float32)
    h2 = lax.dot_general(xg, w2, dn, preferred_element_type=jnp.float32)
    h = ((h1 * jax.nn.sigmoid(h1)) * h2).astype(jnp.bfloat16)
    eo = lax.dot_general(h, wc, dn, preferred_element_type=jnp.float32)
    y_ref[...] = eo * sw_ref[:, 0:1]


def _expert_mlp(xg, w1, w2, wc, slotw, tile_eid):
    grid_spec = pltpu.PrefetchScalarGridSpec(
        num_scalar_prefetch=1,
        grid=(NTILES,),
        in_specs=[
            pl.BlockSpec((TILE, D), lambda i, eid: (i, 0)),
            pl.BlockSpec((1, H, D), lambda i, eid: (eid[0, i], 0, 0)),
            pl.BlockSpec((1, H, D), lambda i, eid: (eid[0, i], 0, 0)),
            pl.BlockSpec((1, D, H), lambda i, eid: (eid[0, i], 0, 0)),
            pl.BlockSpec((TILE, 128), lambda i, eid: (i, 0)),
        ],
        out_specs=pl.BlockSpec((TILE, D), lambda i, eid: (i, 0)),
    )
    return pl.pallas_call(
        _expert_body,
        grid_spec=grid_spec,
        out_shape=jax.ShapeDtypeStruct((NSLOTS, D), jnp.float32),
        compiler_params=pltpu.CompilerParams(
            dimension_semantics=("arbitrary",),
        ),
    )(tile_eid, xg, w1, w2, wc, slotw)


def kernel(x, W1, W2, Wc, Wg):
    b, s, d = x.shape
    x_flat = x.reshape(T, D)
    wg_pad = jnp.zeros((D, 128), jnp.float32).at[:, :NE].set(Wg.T)
    tri = (jnp.arange(128)[:, None] < jnp.arange(128)[None, :]
           ).astype(jnp.float32)
    ids, scores, cnts, offs, te = _router(x_flat, wg_pad, tri)
    pos, xg, slotw = _dispatch_kernel_build()(ids.T.reshape(-1), cnts, offs,
                                              scores.T.reshape(-1), x_flat)
    y = _expert_mlp(xg, W1, W2, Wc, slotw, te)
    out = _combine_kernel_build()(y, pos)
    return out.reshape(b, s, d)


# speedup vs baseline: 1.2141x; 1.0003x over previous
import functools

import jax
import jax.numpy as jnp
from jax import lax
from jax.experimental import pallas as pl
from jax.experimental.pallas import tpu as pltpu
from jax.experimental.pallas import tpu_sc as plsc

T = 2048
D = 1024
H = 1024
NE = 8
K = 2
PAIRS = T * K
TILE = 256
NSLOTS = 6144
NTILES = NSLOTS // TILE


NCHUNK = 32
TOK_PER_CHUNK = T // NCHUNK


def _router_body(x_ref, wg_ref, tri_ref, ids_ref, sc_ref, cnt_ref, offs_ref,
                 te_ref):
    x = x_ref[...]
    wg = wg_ref[...]
    logits = jnp.dot(x, wg, preferred_element_type=jnp.float32)
    lane = lax.broadcasted_iota(jnp.int32, logits.shape, 1)
    neg = jnp.float32(-1e30)
    logits = jnp.where(lane < NE, logits, neg)
    m1 = jnp.max(logits, axis=1, keepdims=True)
    i1 = jnp.min(jnp.where(logits == m1, lane, 128), axis=1, keepdims=True)
    l2 = jnp.where(lane == i1, neg, logits)
    m2 = jnp.max(l2, axis=1, keepdims=True)
    i2 = jnp.min(jnp.where(l2 == m2, lane, 128), axis=1, keepdims=True)
    z = jnp.sum(jnp.exp(logits - m1), axis=1, keepdims=True)
    s1 = 1.0 / z
    s2 = jnp.exp(m2 - m1) / z
    ids_ref[...] = jnp.concatenate([i1, i2], axis=1)
    sc_ref[...] = jnp.concatenate([s1, s2], axis=1)
    nck = NCHUNK // K
    tpc = T // nck
    lane3 = lax.broadcasted_iota(jnp.int32, (nck, tpc, 128), 2)
    h1 = jnp.sum((lane3 == i1.reshape(nck, tpc, 1)).astype(jnp.int32), axis=1)
    h2 = jnp.sum((lane3 == i2.reshape(nck, tpc, 1)).astype(jnp.int32), axis=1)
    cnts = jnp.concatenate([h1, h2], axis=0)
    cnt_ref[...] = cnts[:, :16]
    totals = jnp.sum(cnts, axis=0, keepdims=True).astype(jnp.float32)
    padded = jnp.floor((totals + (TILE - 1)) / TILE) * TILE
    offs = jnp.dot(padded, tri_ref[...],
                   preferred_element_type=jnp.float32)
    offs_i = offs.astype(jnp.int32)
    offs_ref[...] = offs_i[:, :16]
    lane2 = lax.broadcasted_iota(jnp.int32, (1, 128), 1)
    te = jnp.zeros((1, 128), jnp.int32) - 1
    for e in range(NE):
        tstart_e = offs_i[0, e] // TILE
        te = te + (lane2 >= tstart_e).astype(jnp.int32)
    te_ref[...] = te


def _router(x_flat, wg_pad, tri):
    return pl.pallas_call(
        _router_body,
        out_shape=(
            jax.ShapeDtypeStruct((T, K), jnp.int32),
            jax.ShapeDtypeStruct((T, K), jnp.float32),
            jax.ShapeDtypeStruct((NCHUNK, 16), jnp.int32),
            jax.ShapeDtypeStruct((1, 16), jnp.int32),
            jax.ShapeDtypeStruct((1, 128), jnp.int32),
        ),
    )(x_flat, wg_pad, tri)


@functools.cache
def _dispatch_kernel_build():
    mesh = plsc.VectorSubcoreMesh(core_axis_name="c", subcore_axis_name="s", num_cores=2, num_subcores=16)
    return pl.kernel(
        _dispatch_body,
        out_type=(
            jax.ShapeDtypeStruct((PAIRS,), jnp.int32),
            jax.ShapeDtypeStruct((NSLOTS, D), jnp.float32),
            jax.ShapeDtypeStruct((NSLOTS, 128), jnp.float32),
        ),
        mesh=mesh,
        scratch_types=[
            pltpu.VMEM((NCHUNK, 16), jnp.int32),
            pltpu.VMEM((1, 16), jnp.int32),
            pltpu.VMEM((128,), jnp.int32),
            pltpu.VMEM((128,), jnp.float32),
            pltpu.VMEM((4, 32), jnp.int32),
            pltpu.VMEM((4, 32, 128), jnp.float32),
            pltpu.VMEM((2, 32, D), jnp.float32),
            pltpu.SemaphoreType.DMA,
            pltpu.SemaphoreType.DMA,
        ],
        compiler_params=pltpu.CompilerParams(needs_layout_passes=False),
    )


def _dispatch_body(ids_hbm, cnts_hbm, offs_hbm, sc_hbm, x_hbm,
                   pos_hbm, xg_hbm, slotw_hbm,
                   cbuf, offbuf, idv, scv, posbuf, swbuf, rows, gsem, ssem):
    w = lax.axis_index("s") * 2 + lax.axis_index("c")
    tokbase = (w % (NCHUNK // K)) * 128
    pltpu.sync_copy(cnts_hbm, cbuf)
    pltpu.sync_copy(offs_hbm, offbuf)
    pltpu.sync_copy(ids_hbm.at[pl.ds(w * 128, 128)], idv)
    pltpu.sync_copy(sc_hbm.at[pl.ds(w * 128, 128)], scv)
    g = [None] * 4
    sc = [None] * 4
    for s4 in range(2):
        g[s4] = pltpu.async_copy(x_hbm.at[pl.ds(tokbase + s4 * 32, 32)],
                                 rows.at[s4], gsem)
    lane = lax.iota(jnp.int32, 16)
    zero = jnp.zeros((16,), jnp.int32)
    basev = offbuf[0, :]
    for t in range(NCHUNK):
        pred = jnp.where(t < w, 1, 0).astype(jnp.int32)
        basev = basev + cbuf[t, :] * pred
    for s4 in range(4):
        for h in range(2):
            vidx = s4 * 32 + h * 16
            v = idv[pl.ds(vidx, 16)]
            pos_v = zero
            hist = zero
            for e in range(NE):
                m = v == e
                inc = plsc.cumsum(jnp.where(m, 1, 0).astype(jnp.int32))
                pos_v = jnp.where(m, basev[e] + inc - 1, pos_v)
                pc = plsc.all_reduce_population_count(m)
                hist = jnp.where(lane == e, pc, hist)
            basev = basev + hist
            posbuf[s4, pl.ds(h * 16, 16)] = pos_v
            sv = scv[pl.ds(vidx, 16)]
            for j in range(16):
                swbuf[s4, h * 16 + j, pl.ds(0, 16)] = (
                    jnp.zeros((16,), jnp.float32) + sv[j])
        pltpu.sync_copy(posbuf.at[s4],
                        pos_hbm.at[pl.ds(w * 128 + s4 * 32, 32)])
    for s4 in range(4):
        g[s4].wait()
        sc[s4] = pltpu.async_copy(rows.at[s4 % 2], xg_hbm.at[posbuf.at[s4]],
                                  ssem)
        pltpu.async_copy(swbuf.at[s4], slotw_hbm.at[posbuf.at[s4]],
                         ssem).wait()
        if s4 + 2 < 4:
            sc[s4].wait()
            g[s4 + 2] = pltpu.async_copy(
                x_hbm.at[pl.ds(tokbase + (s4 + 2) * 32, 32)],
                rows.at[s4 % 2], gsem)
    sc[2].wait()
    sc[3].wait()


@functools.cache
def _combine_kernel_build():
    mesh = plsc.VectorSubcoreMesh(core_axis_name="c", subcore_axis_name="s", num_cores=2, num_subcores=16)
    return pl.kernel(
        _combine_body,
        out_type=jax.ShapeDtypeStruct((T, D), jnp.float32),
        mesh=mesh,
        scratch_types=[
            pltpu.VMEM((8, 16), jnp.int32),
            pltpu.VMEM((2, 16, D), jnp.float32),
            pltpu.VMEM((2, 16, D), jnp.float32),
            pltpu.SemaphoreType.DMA,
            pltpu.SemaphoreType.DMA,
        ],
        compiler_params=pltpu.CompilerParams(needs_layout_passes=False),
    )


def _combine_body(y_hbm, pos_hbm, out_hbm, posbuf, abuf, bbuf, gsem, osem):
    w = lax.axis_index("s") * 2 + lax.axis_index("c")
    for s4 in range(4):
        pltpu.sync_copy(pos_hbm.at[pl.ds(w * 64 + s4 * 16, 16)],
                        posbuf.at[s4])
        pltpu.sync_copy(pos_hbm.at[pl.ds(T + w * 64 + s4 * 16, 16)],
                        posbuf.at[4 + s4])

    def gather(s4):
        bi = s4 % 2
        ga = pltpu.async_copy(y_hbm.at[posbuf.at[s4]], abuf.at[bi], gsem)
        gb = pltpu.async_copy(y_hbm.at[posbuf.at[4 + s4]], bbuf.at[bi], gsem)
        return ga, gb

    g = [None] * 4
    osc = [None] * 4
    g[0] = gather(0)
    for s4 in range(4):
        bi = s4 % 2
        if s4 + 1 < 4:
            if s4 >= 1:
                osc[s4 - 1].wait()
            g[s4 + 1] = gather(s4 + 1)
        g[s4][0].wait()
        g[s4][1].wait()

        @plsc.parallel_loop(0, 16 * (D // 16), 1, unroll=4)
        def _add(i, bi=bi):
            r = lax.shift_right_logical(i, 6)
            c = lax.shift_left(jnp.bitwise_and(i, D // 16 - 1), 4)
            a = abuf[bi, r, pl.ds(c, 16)]
            bv = bbuf[bi, r, pl.ds(c, 16)]
            abuf[bi, r, pl.ds(c, 16)] = a + bv

        osc[s4] = pltpu.async_copy(
            abuf.at[bi], out_hbm.at[pl.ds(w * 64 + s4 * 16, 16)], osem)
    osc[2].wait()
    osc[3].wait()


def _expert_body(eid_ref, xg_ref, w1_ref, w2_ref, wc_ref, sw_ref, y_ref):
    xg = xg_ref[...]
    w1 = w1_ref[0]
    w2 = w2_ref[0]
    wc = wc_ref[0]
    dn = (((1,), (1,)), ((), ()))
    h1 = lax.dot_general(xg, w1, dn, preferred_element_type=jnp.float32)
    h2 = lax.dot_general(xg, w2, dn, preferred_element_type=jnp.float32)
    h = (h1 * jax.nn.sigmoid(h1)) * h2
    eo = lax.dot_general(h, wc, dn, preferred_element_type=jnp.float32)
    y_ref[...] = eo * sw_ref[:, 0:1]


def _expert_mlp(xg, w1, w2, wc, slotw, tile_eid):
    grid_spec = pltpu.PrefetchScalarGridSpec(
        num_scalar_prefetch=1,
        grid=(NTILES,),
        in_specs=[
            pl.BlockSpec((TILE, D), lambda i, eid: (i, 0)),
            pl.BlockSpec((1, H, D), lambda i, eid: (eid[0, i], 0, 0)),
            pl.BlockSpec((1, H, D), lambda i, eid: (eid[0, i], 0, 0)),
            pl.BlockSpec((1, D, H), lambda i, eid: (eid[0, i], 0, 0)),
            pl.BlockSpec((TILE, 128), lambda i, eid: (i, 0)),
        ],
        out_specs=pl.BlockSpec((TILE, D), lambda i, eid: (i, 0)),
    )
    return pl.pallas_call(
        _expert_body,
        grid_spec=grid_spec,
        out_shape=jax.ShapeDtypeStruct((NSLOTS, D), jnp.float32),
        compiler_params=pltpu.CompilerParams(
            dimension_semantics=("arbitrary",),
        ),
    )(tile_eid, xg, w1, w2, wc, slotw)


def kernel(x, W1, W2, Wc, Wg):
    b, s, d = x.shape
    x_flat = x.reshape(T, D)
    wg_pad = jnp.zeros((D, 128), jnp.float32).at[:, :NE].set(Wg.T)
    tri = (jnp.arange(128)[:, None] < jnp.arange(128)[None, :]
           ).astype(jnp.float32)
    ids, scores, cnts, offs, te = _router(x_flat, wg_pad, tri)
    pos, xg, slotw = _dispatch_kernel_build()(ids.T.reshape(-1), cnts, offs,
                                              scores.T.reshape(-1), x_flat)
    y = _expert_mlp(xg, W1, W2, Wc, slotw, te)
    out = _combine_kernel_build()(y, pos)
    return out.reshape(b, s, d)


# speedup vs baseline: 1.2208x; 1.0056x over previous
import functools

import jax
import jax.numpy as jnp
from jax import lax
from jax.experimental import pallas as pl
from jax.experimental.pallas import tpu as pltpu
from jax.experimental.pallas import tpu_sc as plsc

T = 2048
D = 1024
H = 1024
NE = 8
K = 2
PAIRS = T * K
TILE = 256
NSLOTS = 6144
NTILES = NSLOTS // TILE


NCHUNK = 32
TOK_PER_CHUNK = T // NCHUNK


def _router_body(x_ref, wg_ref, tri_ref, ids_ref, sc_ref, cnt_ref, offs_ref,
                 te_ref):
    x = x_ref[...]
    wg = wg_ref[...]
    logits = jnp.dot(x, wg, preferred_element_type=jnp.float32)
    lane = lax.broadcasted_iota(jnp.int32, logits.shape, 1)
    neg = jnp.float32(-1e30)
    logits = jnp.where(lane < NE, logits, neg)
    m1 = jnp.max(logits, axis=1, keepdims=True)
    i1 = jnp.min(jnp.where(logits == m1, lane, 128), axis=1, keepdims=True)
    l2 = jnp.where(lane == i1, neg, logits)
    m2 = jnp.max(l2, axis=1, keepdims=True)
    i2 = jnp.min(jnp.where(l2 == m2, lane, 128), axis=1, keepdims=True)
    z = jnp.sum(jnp.exp(logits - m1), axis=1, keepdims=True)
    s1 = 1.0 / z
    s2 = jnp.exp(m2 - m1) / z
    ids_ref[...] = jnp.concatenate(
        [jnp.transpose(i1), jnp.transpose(i2)], axis=0)
    sc_ref[...] = jnp.concatenate(
        [jnp.transpose(s1), jnp.transpose(s2)], axis=0)
    nck = NCHUNK // K
    tpc = T // nck
    lane3 = lax.broadcasted_iota(jnp.int32, (nck, tpc, 128), 2)
    h1 = jnp.sum((lane3 == i1.reshape(nck, tpc, 1)).astype(jnp.int32), axis=1)
    h2 = jnp.sum((lane3 == i2.reshape(nck, tpc, 1)).astype(jnp.int32), axis=1)
    cnts = jnp.concatenate([h1, h2], axis=0)
    cnt_ref[...] = cnts[:, :16]
    totals = jnp.sum(cnts, axis=0, keepdims=True).astype(jnp.float32)
    padded = jnp.floor((totals + (TILE - 1)) / TILE) * TILE
    offs = jnp.dot(padded, tri_ref[...],
                   preferred_element_type=jnp.float32)
    offs_i = offs.astype(jnp.int32)
    offs_ref[...] = offs_i[:, :16]
    lane2 = lax.broadcasted_iota(jnp.int32, (1, 128), 1)
    te = jnp.zeros((1, 128), jnp.int32) - 1
    for e in range(NE):
        tstart_e = offs_i[0, e] // TILE
        te = te + (lane2 >= tstart_e).astype(jnp.int32)
    te_ref[...] = te


def _router(x_flat, wg_pad, tri):
    return pl.pallas_call(
        _router_body,
        out_shape=(
            jax.ShapeDtypeStruct((K, T), jnp.int32),
            jax.ShapeDtypeStruct((K, T), jnp.float32),
            jax.ShapeDtypeStruct((NCHUNK, 16), jnp.int32),
            jax.ShapeDtypeStruct((1, 16), jnp.int32),
            jax.ShapeDtypeStruct((1, 128), jnp.int32),
        ),
    )(x_flat, wg_pad, tri)


@functools.cache
def _dispatch_kernel_build():
    mesh = plsc.VectorSubcoreMesh(core_axis_name="c", subcore_axis_name="s", num_cores=2, num_subcores=16)
    return pl.kernel(
        _dispatch_body,
        out_type=(
            jax.ShapeDtypeStruct((PAIRS,), jnp.int32),
            jax.ShapeDtypeStruct((NSLOTS, D), jnp.float32),
            jax.ShapeDtypeStruct((NSLOTS, 128), jnp.float32),
        ),
        mesh=mesh,
        scratch_types=[
            pltpu.VMEM((NCHUNK, 16), jnp.int32),
            pltpu.VMEM((1, 16), jnp.int32),
            pltpu.VMEM((128,), jnp.int32),
            pltpu.VMEM((128,), jnp.float32),
            pltpu.VMEM((4, 32), jnp.int32),
            pltpu.VMEM((4, 32, 128), jnp.float32),
            pltpu.VMEM((2, 32, D), jnp.float32),
            pltpu.SemaphoreType.DMA,
            pltpu.SemaphoreType.DMA,
        ],
        compiler_params=pltpu.CompilerParams(needs_layout_passes=False),
    )


def _dispatch_body(ids_hbm, cnts_hbm, offs_hbm, sc_hbm, x_hbm,
                   pos_hbm, xg_hbm, slotw_hbm,
                   cbuf, offbuf, idv, scv, posbuf, swbuf, rows, gsem, ssem):
    w = lax.axis_index("s") * 2 + lax.axis_index("c")
    tokbase = (w % (NCHUNK // K)) * 128
    pltpu.sync_copy(cnts_hbm, cbuf)
    pltpu.sync_copy(offs_hbm, offbuf)
    pltpu.sync_copy(ids_hbm.at[pl.ds(w * 128, 128)], idv)
    pltpu.sync_copy(sc_hbm.at[pl.ds(w * 128, 128)], scv)
    g = [None] * 4
    sc = [None] * 4
    for s4 in range(2):
        g[s4] = pltpu.async_copy(x_hbm.at[pl.ds(tokbase + s4 * 32, 32)],
                                 rows.at[s4], gsem)
    lane = lax.iota(jnp.int32, 16)
    zero = jnp.zeros((16,), jnp.int32)
    basev = offbuf[0, :]
    for t in range(NCHUNK):
        pred = jnp.where(t < w, 1, 0).astype(jnp.int32)
        basev = basev + cbuf[t, :] * pred
    for s4 in range(4):
        for h in range(2):
            vidx = s4 * 32 + h * 16
            v = idv[pl.ds(vidx, 16)]
            pos_v = zero
            hist = zero
            for e in range(NE):
                m = v == e
                inc = plsc.cumsum(jnp.where(m, 1, 0).astype(jnp.int32))
                pos_v = jnp.where(m, basev[e] + inc - 1, pos_v)
                pc = plsc.all_reduce_population_count(m)
                hist = jnp.where(lane == e, pc, hist)
            basev = basev + hist
            posbuf[s4, pl.ds(h * 16, 16)] = pos_v
            sv = scv[pl.ds(vidx, 16)]
            for j in range(16):
                swbuf[s4, h * 16 + j, pl.ds(0, 16)] = (
                    jnp.zeros((16,), jnp.float32) + sv[j])
        pltpu.sync_copy(posbuf.at[s4],
                        pos_hbm.at[pl.ds(w * 128 + s4 * 32, 32)])
    for s4 in range(4):
        g[s4].wait()
        sc[s4] = pltpu.async_copy(rows.at[s4 % 2], xg_hbm.at[posbuf.at[s4]],
                                  ssem)
        pltpu.async_copy(swbuf.at[s4], slotw_hbm.at[posbuf.at[s4]],
                         ssem).wait()
        if s4 + 2 < 4:
            sc[s4].wait()
            g[s4 + 2] = pltpu.async_copy(
                x_hbm.at[pl.ds(tokbase + (s4 + 2) * 32, 32)],
                rows.at[s4 % 2], gsem)
    sc[2].wait()
    sc[3].wait()


@functools.cache
def _combine_kernel_build():
    mesh = plsc.VectorSubcoreMesh(core_axis_name="c", subcore_axis_name="s", num_cores=2, num_subcores=16)
    return pl.kernel(
        _combine_body,
        out_type=jax.ShapeDtypeStruct((T, D), jnp.float32),
        mesh=mesh,
        scratch_types=[
            pltpu.VMEM((8, 16), jnp.int32),
            pltpu.VMEM((2, 16, D), jnp.float32),
            pltpu.VMEM((2, 16, D), jnp.float32),
            pltpu.SemaphoreType.DMA,
            pltpu.SemaphoreType.DMA,
        ],
        compiler_params=pltpu.CompilerParams(needs_layout_passes=False),
    )


def _combine_body(y_hbm, pos_hbm, out_hbm, posbuf, abuf, bbuf, gsem, osem):
    w = lax.axis_index("s") * 2 + lax.axis_index("c")
    for s4 in range(4):
        pltpu.sync_copy(pos_hbm.at[pl.ds(w * 64 + s4 * 16, 16)],
                        posbuf.at[s4])
        pltpu.sync_copy(pos_hbm.at[pl.ds(T + w * 64 + s4 * 16, 16)],
                        posbuf.at[4 + s4])

    def gather(s4):
        bi = s4 % 2
        ga = pltpu.async_copy(y_hbm.at[posbuf.at[s4]], abuf.at[bi], gsem)
        gb = pltpu.async_copy(y_hbm.at[posbuf.at[4 + s4]], bbuf.at[bi], gsem)
        return ga, gb

    g = [None] * 4
    osc = [None] * 4
    g[0] = gather(0)
    for s4 in range(4):
        bi = s4 % 2
        if s4 + 1 < 4:
            if s4 >= 1:
                osc[s4 - 1].wait()
            g[s4 + 1] = gather(s4 + 1)
        g[s4][0].wait()
        g[s4][1].wait()

        @plsc.parallel_loop(0, 16 * (D // 16), 1, unroll=4)
        def _add(i, bi=bi):
            r = lax.shift_right_logical(i, 6)
            c = lax.shift_left(jnp.bitwise_and(i, D // 16 - 1), 4)
            a = abuf[bi, r, pl.ds(c, 16)]
            bv = bbuf[bi, r, pl.ds(c, 16)]
            abuf[bi, r, pl.ds(c, 16)] = a + bv

        osc[s4] = pltpu.async_copy(
            abuf.at[bi], out_hbm.at[pl.ds(w * 64 + s4 * 16, 16)], osem)
    osc[2].wait()
    osc[3].wait()


def _expert_body(eid_ref, xg_ref, w1_ref, w2_ref, wc_ref, sw_ref, y_ref):
    xg = xg_ref[...]
    w1 = w1_ref[0]
    w2 = w2_ref[0]
    wc = wc_ref[0]
    dn = (((1,), (1,)), ((), ()))
    h1 = lax.dot_general(xg, w1, dn, preferred_element_type=jnp.float32)
    h2 = lax.dot_general(xg, w2, dn, preferred_element_type=jnp.float32)
    h = (h1 * jax.nn.sigmoid(h1)) * h2
    eo = lax.dot_general(h, wc, dn, preferred_element_type=jnp.float32)
    y_ref[...] = eo * sw_ref[:, 0:1]


def _expert_mlp(xg, w1, w2, wc, slotw, tile_eid):
    grid_spec = pltpu.PrefetchScalarGridSpec(
        num_scalar_prefetch=1,
        grid=(NTILES,),
        in_specs=[
            pl.BlockSpec((TILE, D), lambda i, eid: (i, 0)),
            pl.BlockSpec((1, H, D), lambda i, eid: (eid[0, i], 0, 0)),
            pl.BlockSpec((1, H, D), lambda i, eid: (eid[0, i], 0, 0)),
            pl.BlockSpec((1, D, H), lambda i, eid: (eid[0, i], 0, 0)),
            pl.BlockSpec((TILE, 128), lambda i, eid: (i, 0)),
        ],
        out_specs=pl.BlockSpec((TILE, D), lambda i, eid: (i, 0)),
    )
    return pl.pallas_call(
        _expert_body,
        grid_spec=grid_spec,
        out_shape=jax.ShapeDtypeStruct((NSLOTS, D), jnp.float32),
        compiler_params=pltpu.CompilerParams(
            dimension_semantics=("arbitrary",),
        ),
    )(tile_eid, xg, w1, w2, wc, slotw)


def kernel(x, W1, W2, Wc, Wg):
    b, s, d = x.shape
    x_flat = x.reshape(T, D)
    wg_pad = jnp.zeros((D, 128), jnp.float32).at[:, :NE].set(Wg.T)
    tri = (jnp.arange(128)[:, None] < jnp.arange(128)[None, :]
           ).astype(jnp.float32)
    ids, scores, cnts, offs, te = _router(x_flat, wg_pad, tri)
    pos, xg, slotw = _dispatch_kernel_build()(ids.reshape(-1), cnts, offs,
                                              scores.reshape(-1), x_flat)
    y = _expert_mlp(xg, W1, W2, Wc, slotw, te)
    out = _combine_kernel_build()(y, pos)
    return out.reshape(b, s, d)


# speedup vs baseline: 1.2457x; 1.0204x over previous
import functools

import jax
import jax.numpy as jnp
from jax import lax
from jax.experimental import pallas as pl
from jax.experimental.pallas import tpu as pltpu
from jax.experimental.pallas import tpu_sc as plsc

T = 2048
D = 1024
H = 1024
NE = 8
K = 2
PAIRS = T * K
TILE = 256
NSLOTS = 6144
NTILES = NSLOTS // TILE


NCHUNK = 32
TOK_PER_CHUNK = T // NCHUNK


def _router_body(x_ref, wg_ref, tri_ref, ids_ref, sc_ref, cnt_ref, offs_ref,
                 te_ref):
    x = x_ref[...]
    wg = wg_ref[...]
    logits = jnp.dot(x, wg, preferred_element_type=jnp.float32)
    lane = lax.broadcasted_iota(jnp.int32, logits.shape, 1)
    neg = jnp.float32(-1e30)
    logits = jnp.where(lane < NE, logits, neg)
    m1 = jnp.max(logits, axis=1, keepdims=True)
    i1 = jnp.min(jnp.where(logits == m1, lane, 128), axis=1, keepdims=True)
    l2 = jnp.where(lane == i1, neg, logits)
    m2 = jnp.max(l2, axis=1, keepdims=True)
    i2 = jnp.min(jnp.where(l2 == m2, lane, 128), axis=1, keepdims=True)
    z = jnp.sum(jnp.exp(logits - m1), axis=1, keepdims=True)
    s1 = 1.0 / z
    s2 = jnp.exp(m2 - m1) / z
    ids_ref[...] = jnp.concatenate(
        [jnp.transpose(i1), jnp.transpose(i2)], axis=1).reshape(PAIRS)
    sc_ref[...] = jnp.concatenate(
        [jnp.transpose(s1), jnp.transpose(s2)], axis=1).reshape(PAIRS)
    nck = NCHUNK // K
    tpc = T // nck
    lane3 = lax.broadcasted_iota(jnp.int32, (nck, tpc, 128), 2)
    h1 = jnp.sum((lane3 == i1.reshape(nck, tpc, 1)).astype(jnp.int32), axis=1)
    h2 = jnp.sum((lane3 == i2.reshape(nck, tpc, 1)).astype(jnp.int32), axis=1)
    cnts = jnp.concatenate([h1, h2], axis=0)
    cnt_ref[...] = cnts[:, :16]
    totals = jnp.sum(cnts, axis=0, keepdims=True).astype(jnp.float32)
    padded = jnp.floor((totals + (TILE - 1)) / TILE) * TILE
    offs = jnp.dot(padded, tri_ref[...],
                   preferred_element_type=jnp.float32)
    offs_i = offs.astype(jnp.int32)
    offs_ref[...] = offs_i[:, :16]
    lane2 = lax.broadcasted_iota(jnp.int32, (1, 128), 1)
    te = jnp.zeros((1, 128), jnp.int32) - 1
    for e in range(NE):
        tstart_e = offs_i[0, e] // TILE
        te = te + (lane2 >= tstart_e).astype(jnp.int32)
    te_ref[...] = te


def _router(x_flat, wg_pad, tri):
    return pl.pallas_call(
        _router_body,
        out_shape=(
            jax.ShapeDtypeStruct((PAIRS,), jnp.int32),
            jax.ShapeDtypeStruct((PAIRS,), jnp.float32),
            jax.ShapeDtypeStruct((NCHUNK, 16), jnp.int32),
            jax.ShapeDtypeStruct((1, 16), jnp.int32),
            jax.ShapeDtypeStruct((1, 128), jnp.int32),
        ),
    )(x_flat, wg_pad, tri)


@functools.cache
def _dispatch_kernel_build():
    mesh = plsc.VectorSubcoreMesh(core_axis_name="c", subcore_axis_name="s", num_cores=2, num_subcores=16)
    return pl.kernel(
        _dispatch_body,
        out_type=(
            jax.ShapeDtypeStruct((PAIRS,), jnp.int32),
            jax.ShapeDtypeStruct((NSLOTS, D), jnp.float32),
            jax.ShapeDtypeStruct((NSLOTS, 128), jnp.float32),
        ),
        mesh=mesh,
        scratch_types=[
            pltpu.VMEM((NCHUNK, 16), jnp.int32),
            pltpu.VMEM((1, 16), jnp.int32),
            pltpu.VMEM((128,), jnp.int32),
            pltpu.VMEM((128,), jnp.float32),
            pltpu.VMEM((4, 32), jnp.int32),
            pltpu.VMEM((4, 32, 128), jnp.float32),
            pltpu.VMEM((2, 32, D), jnp.float32),
            pltpu.SemaphoreType.DMA,
            pltpu.SemaphoreType.DMA,
        ],
        compiler_params=pltpu.CompilerParams(needs_layout_passes=False),
    )


def _dispatch_body(ids_hbm, cnts_hbm, offs_hbm, sc_hbm, x_hbm,
                   pos_hbm, xg_hbm, slotw_hbm,
                   cbuf, offbuf, idv, scv, posbuf, swbuf, rows, gsem, ssem):
    w = lax.axis_index("s") * 2 + lax.axis_index("c")
    tokbase = (w % (NCHUNK // K)) * 128
    pltpu.sync_copy(cnts_hbm, cbuf)
    pltpu.sync_copy(offs_hbm, offbuf)
    pltpu.sync_copy(ids_hbm.at[pl.ds(w * 128, 128)], idv)
    pltpu.sync_copy(sc_hbm.at[pl.ds(w * 128, 128)], scv)
    g = [None] * 4
    sc = [None] * 4
    for s4 in range(2):
        g[s4] = pltpu.async_copy(x_hbm.at[pl.ds(tokbase + s4 * 32, 32)],
                                 rows.at[s4], gsem)
    lane = lax.iota(jnp.int32, 16)
    zero = jnp.zeros((16,), jnp.int32)
    basev = offbuf[0, :]
    for t in range(NCHUNK):
        pred = jnp.where(t < w, 1, 0).astype(jnp.int32)
        basev = basev + cbuf[t, :] * pred
    for s4 in range(4):
        for h in range(2):
            vidx = s4 * 32 + h * 16
            v = idv[pl.ds(vidx, 16)]
            pos_v = zero
            hist = zero
            for e in range(NE):
                m = v == e
                inc = plsc.cumsum(jnp.where(m, 1, 0).astype(jnp.int32))
                pos_v = jnp.where(m, basev[e] + inc - 1, pos_v)
                pc = plsc.all_reduce_population_count(m)
                hist = jnp.where(lane == e, pc, hist)
            basev = basev + hist
            posbuf[s4, pl.ds(h * 16, 16)] = pos_v
            sv = scv[pl.ds(vidx, 16)]
            for j in range(16):
                swbuf[s4, h * 16 + j, pl.ds(0, 16)] = (
                    jnp.zeros((16,), jnp.float32) + sv[j])
        pltpu.sync_copy(posbuf.at[s4],
                        pos_hbm.at[pl.ds(w * 128 + s4 * 32, 32)])
    for s4 in range(4):
        g[s4].wait()
        sc[s4] = pltpu.async_copy(rows.at[s4 % 2], xg_hbm.at[posbuf.at[s4]],
                                  ssem)
        pltpu.async_copy(swbuf.at[s4], slotw_hbm.at[posbuf.at[s4]],
                         ssem).wait()
        if s4 + 2 < 4:
            sc[s4].wait()
            g[s4 + 2] = pltpu.async_copy(
                x_hbm.at[pl.ds(tokbase + (s4 + 2) * 32, 32)],
                rows.at[s4 % 2], gsem)
    sc[2].wait()
    sc[3].wait()


@functools.cache
def _combine_kernel_build():
    mesh = plsc.VectorSubcoreMesh(core_axis_name="c", subcore_axis_name="s", num_cores=2, num_subcores=16)
    return pl.kernel(
        _combine_body,
        out_type=jax.ShapeDtypeStruct((T, D), jnp.float32),
        mesh=mesh,
        scratch_types=[
            pltpu.VMEM((8, 16), jnp.int32),
            pltpu.VMEM((2, 16, D), jnp.float32),
            pltpu.VMEM((2, 16, D), jnp.float32),
            pltpu.SemaphoreType.DMA,
            pltpu.SemaphoreType.DMA,
        ],
        compiler_params=pltpu.CompilerParams(needs_layout_passes=False),
    )


def _combine_body(y_hbm, pos_hbm, out_hbm, posbuf, abuf, bbuf, gsem, osem):
    w = lax.axis_index("s") * 2 + lax.axis_index("c")
    for s4 in range(4):
        pltpu.sync_copy(pos_hbm.at[pl.ds(w * 64 + s4 * 16, 16)],
                        posbuf.at[s4])
        pltpu.sync_copy(pos_hbm.at[pl.ds(T + w * 64 + s4 * 16, 16)],
                        posbuf.at[4 + s4])

    def gather(s4):
        bi = s4 % 2
        ga = pltpu.async_copy(y_hbm.at[posbuf.at[s4]], abuf.at[bi], gsem)
        gb = pltpu.async_copy(y_hbm.at[posbuf.at[4 + s4]], bbuf.at[bi], gsem)
        return ga, gb

    g = [None] * 4
    osc = [None] * 4
    g[0] = gather(0)
    for s4 in range(4):
        bi = s4 % 2
        if s4 + 1 < 4:
            if s4 >= 1:
                osc[s4 - 1].wait()
            g[s4 + 1] = gather(s4 + 1)
        g[s4][0].wait()
        g[s4][1].wait()

        @plsc.parallel_loop(0, 16 * (D // 16), 1, unroll=4)
        def _add(i, bi=bi):
            r = lax.shift_right_logical(i, 6)
            c = lax.shift_left(jnp.bitwise_and(i, D // 16 - 1), 4)
            a = abuf[bi, r, pl.ds(c, 16)]
            bv = bbuf[bi, r, pl.ds(c, 16)]
            abuf[bi, r, pl.ds(c, 16)] = a + bv

        osc[s4] = pltpu.async_copy(
            abuf.at[bi], out_hbm.at[pl.ds(w * 64 + s4 * 16, 16)], osem)
    osc[2].wait()
    osc[3].wait()


def _expert_body(eid_ref, xg_ref, w1_ref, w2_ref, wc_ref, sw_ref, y_ref):
    xg = xg_ref[...]
    w1 = w1_ref[0]
    w2 = w2_ref[0]
    wc = wc_ref[0]
    dn = (((1,), (1,)), ((), ()))
    h1 = lax.dot_general(xg, w1, dn, preferred_element_type=jnp.float32)
    h2 = lax.dot_general(xg, w2, dn, preferred_element_type=jnp.float32)
    h = (h1 * jax.nn.sigmoid(h1)) * h2
    eo = lax.dot_general(h, wc, dn, preferred_element_type=jnp.float32)
    y_ref[...] = eo * sw_ref[:, 0:1]


def _expert_mlp(xg, w1, w2, wc, slotw, tile_eid):
    grid_spec = pltpu.PrefetchScalarGridSpec(
        num_scalar_prefetch=1,
        grid=(NTILES,),
        in_specs=[
            pl.BlockSpec((TILE, D), lambda i, eid: (i, 0)),
            pl.BlockSpec((1, H, D), lambda i, eid: (eid[0, i], 0, 0)),
            pl.BlockSpec((1, H, D), lambda i, eid: (eid[0, i], 0, 0)),
            pl.BlockSpec((1, D, H), lambda i, eid: (eid[0, i], 0, 0)),
            pl.BlockSpec((TILE, 128), lambda i, eid: (i, 0)),
        ],
        out_specs=pl.BlockSpec((TILE, D), lambda i, eid: (i, 0)),
    )
    return pl.pallas_call(
        _expert_body,
        grid_spec=grid_spec,
        out_shape=jax.ShapeDtypeStruct((NSLOTS, D), jnp.float32),
        compiler_params=pltpu.CompilerParams(
            dimension_semantics=("arbitrary",),
        ),
    )(tile_eid, xg, w1, w2, wc, slotw)


def kernel(x, W1, W2, Wc, Wg):
    b, s, d = x.shape
    x_flat = x.reshape(T, D)
    wg_pad = jnp.zeros((D, 128), jnp.float32).at[:, :NE].set(Wg.T)
    tri = (jnp.arange(128)[:, None] < jnp.arange(128)[None, :]
           ).astype(jnp.float32)
    ids, scores, cnts, offs, te = _router(x_flat, wg_pad, tri)
    pos, xg, slotw = _dispatch_kernel_build()(ids, cnts, offs, scores, x_flat)
    y = _expert_mlp(xg, W1, W2, Wc, slotw, te)
    out = _combine_kernel_build()(y, pos)
    return out.reshape(b, s, d)


# speedup vs baseline: 1.2491x; 1.0027x over previous
import functools

import jax
import jax.numpy as jnp
from jax import lax
from jax.experimental import pallas as pl
from jax.experimental.pallas import tpu as pltpu
from jax.experimental.pallas import tpu_sc as plsc

T = 2048
D = 1024
H = 1024
NE = 8
K = 2
PAIRS = T * K
TILE = 256
NSLOTS = 6144
NTILES = NSLOTS // TILE


NCHUNK = 32
TOK_PER_CHUNK = T // NCHUNK


def _router_body(x_ref, wg_ref, tri_ref, ids_ref, sc_ref, cnt_ref, offs_ref,
                 te_ref):
    x = x_ref[...]
    wg = wg_ref[...]
    logits = jnp.dot(x, wg, preferred_element_type=jnp.float32)
    lane = lax.broadcasted_iota(jnp.int32, logits.shape, 1)
    neg = jnp.float32(-1e30)
    logits = jnp.where(lane < NE, logits, neg)
    m1 = jnp.max(logits, axis=1, keepdims=True)
    i1 = jnp.min(jnp.where(logits == m1, lane, 128), axis=1, keepdims=True)
    l2 = jnp.where(lane == i1, neg, logits)
    m2 = jnp.max(l2, axis=1, keepdims=True)
    i2 = jnp.min(jnp.where(l2 == m2, lane, 128), axis=1, keepdims=True)
    z = jnp.sum(jnp.exp(logits - m1), axis=1, keepdims=True)
    s1 = 1.0 / z
    s2 = jnp.exp(m2 - m1) / z
    ids_ref[...] = jnp.concatenate(
        [jnp.transpose(i1), jnp.transpose(i2)], axis=1).reshape(PAIRS)
    sc_ref[...] = jnp.concatenate(
        [jnp.transpose(s1), jnp.transpose(s2)], axis=1).reshape(PAIRS)
    nck = NCHUNK // K
    tpc = T // nck
    lane3 = lax.broadcasted_iota(jnp.int32, (nck, tpc, 128), 2)
    h1 = jnp.sum((lane3 == i1.reshape(nck, tpc, 1)).astype(jnp.int32), axis=1)
    h2 = jnp.sum((lane3 == i2.reshape(nck, tpc, 1)).astype(jnp.int32), axis=1)
    cnts = jnp.concatenate([h1, h2], axis=0)
    cnt_ref[...] = cnts[:, :16]
    totals = jnp.sum(cnts, axis=0, keepdims=True).astype(jnp.float32)
    padded = jnp.floor((totals + (TILE - 1)) / TILE) * TILE
    offs = jnp.dot(padded, tri_ref[...],
                   preferred_element_type=jnp.float32)
    offs_i = offs.astype(jnp.int32)
    offs_ref[...] = offs_i[:, :16].reshape(16)
    lane2 = lax.broadcasted_iota(jnp.int32, (1, 128), 1)
    te = jnp.zeros((1, 128), jnp.int32) - 1
    for e in range(NE):
        tstart_e = offs_i[0, e] // TILE
        te = te + (lane2 >= tstart_e).astype(jnp.int32)
    te_ref[...] = te.reshape(128)


def _router(x_flat, wg_pad, tri):
    return pl.pallas_call(
        _router_body,
        out_shape=(
            jax.ShapeDtypeStruct((PAIRS,), jnp.int32),
            jax.ShapeDtypeStruct((PAIRS,), jnp.float32),
            jax.ShapeDtypeStruct((NCHUNK, 16), jnp.int32),
            jax.ShapeDtypeStruct((16,), jnp.int32),
            jax.ShapeDtypeStruct((128,), jnp.int32),
        ),
    )(x_flat, wg_pad, tri)


@functools.cache
def _dispatch_kernel_build():
    mesh = plsc.VectorSubcoreMesh(core_axis_name="c", subcore_axis_name="s", num_cores=2, num_subcores=16)
    return pl.kernel(
        _dispatch_body,
        out_type=(
            jax.ShapeDtypeStruct((PAIRS,), jnp.int32),
            jax.ShapeDtypeStruct((NSLOTS, D), jnp.float32),
            jax.ShapeDtypeStruct((NSLOTS, 128), jnp.float32),
        ),
        mesh=mesh,
        scratch_types=[
            pltpu.VMEM((NCHUNK, 16), jnp.int32),
            pltpu.VMEM((16,), jnp.int32),
            pltpu.VMEM((128,), jnp.int32),
            pltpu.VMEM((128,), jnp.float32),
            pltpu.VMEM((4, 32), jnp.int32),
            pltpu.VMEM((4, 32, 128), jnp.float32),
            pltpu.VMEM((2, 32, D), jnp.float32),
            pltpu.SemaphoreType.DMA,
            pltpu.SemaphoreType.DMA,
        ],
        compiler_params=pltpu.CompilerParams(needs_layout_passes=False),
    )


def _dispatch_body(ids_hbm, cnts_hbm, offs_hbm, sc_hbm, x_hbm,
                   pos_hbm, xg_hbm, slotw_hbm,
                   cbuf, offbuf, idv, scv, posbuf, swbuf, rows, gsem, ssem):
    w = lax.axis_index("s") * 2 + lax.axis_index("c")
    tokbase = (w % (NCHUNK // K)) * 128
    pltpu.sync_copy(cnts_hbm, cbuf)
    pltpu.sync_copy(offs_hbm, offbuf)
    pltpu.sync_copy(ids_hbm.at[pl.ds(w * 128, 128)], idv)
    pltpu.sync_copy(sc_hbm.at[pl.ds(w * 128, 128)], scv)
    g = [None] * 4
    sc = [None] * 4
    for s4 in range(2):
        g[s4] = pltpu.async_copy(x_hbm.at[pl.ds(tokbase + s4 * 32, 32)],
                                 rows.at[s4], gsem)
    lane = lax.iota(jnp.int32, 16)
    zero = jnp.zeros((16,), jnp.int32)
    basev = offbuf[...]
    for t in range(NCHUNK):
        pred = jnp.where(t < w, 1, 0).astype(jnp.int32)
        basev = basev + cbuf[t, :] * pred
    for s4 in range(4):
        for h in range(2):
            vidx = s4 * 32 + h * 16
            v = idv[pl.ds(vidx, 16)]
            pos_v = zero
            hist = zero
            for e in range(NE):
                m = v == e
                inc = plsc.cumsum(jnp.where(m, 1, 0).astype(jnp.int32))
                pos_v = jnp.where(m, basev[e] + inc - 1, pos_v)
                pc = plsc.all_reduce_population_count(m)
                hist = jnp.where(lane == e, pc, hist)
            basev = basev + hist
            posbuf[s4, pl.ds(h * 16, 16)] = pos_v
            sv = scv[pl.ds(vidx, 16)]
            for j in range(16):
                swbuf[s4, h * 16 + j, pl.ds(0, 16)] = (
                    jnp.zeros((16,), jnp.float32) + sv[j])
        pltpu.sync_copy(posbuf.at[s4],
                        pos_hbm.at[pl.ds(w * 128 + s4 * 32, 32)])
    for s4 in range(4):
        g[s4].wait()
        sc[s4] = pltpu.async_copy(rows.at[s4 % 2], xg_hbm.at[posbuf.at[s4]],
                                  ssem)
        pltpu.async_copy(swbuf.at[s4], slotw_hbm.at[posbuf.at[s4]],
                         ssem).wait()
        if s4 + 2 < 4:
            sc[s4].wait()
            g[s4 + 2] = pltpu.async_copy(
                x_hbm.at[pl.ds(tokbase + (s4 + 2) * 32, 32)],
                rows.at[s4 % 2], gsem)
    sc[2].wait()
    sc[3].wait()


@functools.cache
def _combine_kernel_build():
    mesh = plsc.VectorSubcoreMesh(core_axis_name="c", subcore_axis_name="s", num_cores=2, num_subcores=16)
    return pl.kernel(
        _combine_body,
        out_type=jax.ShapeDtypeStruct((T, D), jnp.float32),
        mesh=mesh,
        scratch_types=[
            pltpu.VMEM((8, 16), jnp.int32),
            pltpu.VMEM((2, 16, D), jnp.float32),
            pltpu.VMEM((2, 16, D), jnp.float32),
            pltpu.SemaphoreType.DMA,
            pltpu.SemaphoreType.DMA,
        ],
        compiler_params=pltpu.CompilerParams(needs_layout_passes=False),
    )


def _combine_body(y_hbm, pos_hbm, out_hbm, posbuf, abuf, bbuf, gsem, osem):
    w = lax.axis_index("s") * 2 + lax.axis_index("c")
    for s4 in range(4):
        pltpu.sync_copy(pos_hbm.at[pl.ds(w * 64 + s4 * 16, 16)],
                        posbuf.at[s4])
        pltpu.sync_copy(pos_hbm.at[pl.ds(T + w * 64 + s4 * 16, 16)],
                        posbuf.at[4 + s4])

    def gather(s4):
        bi = s4 % 2
        ga = pltpu.async_copy(y_hbm.at[posbuf.at[s4]], abuf.at[bi], gsem)
        gb = pltpu.async_copy(y_hbm.at[posbuf.at[4 + s4]], bbuf.at[bi], gsem)
        return ga, gb

    g = [None] * 4
    osc = [None] * 4
    g[0] = gather(0)
    for s4 in range(4):
        bi = s4 % 2
        if s4 + 1 < 4:
            if s4 >= 1:
                osc[s4 - 1].wait()
            g[s4 + 1] = gather(s4 + 1)
        g[s4][0].wait()
        g[s4][1].wait()

        @plsc.parallel_loop(0, 16 * (D // 16), 1, unroll=4)
        def _add(i, bi=bi):
            r = lax.shift_right_logical(i, 6)
            c = lax.shift_left(jnp.bitwise_and(i, D // 16 - 1), 4)
            a = abuf[bi, r, pl.ds(c, 16)]
            bv = bbuf[bi, r, pl.ds(c, 16)]
            abuf[bi, r, pl.ds(c, 16)] = a + bv

        osc[s4] = pltpu.async_copy(
            abuf.at[bi], out_hbm.at[pl.ds(w * 64 + s4 * 16, 16)], osem)
    osc[2].wait()
    osc[3].wait()


def _expert_body(eid_ref, xg_ref, w1_ref, w2_ref, wc_ref, sw_ref, y_ref):
    xg = xg_ref[...]
    w1 = w1_ref[0]
    w2 = w2_ref[0]
    wc = wc_ref[0]
    dn = (((1,), (1,)), ((), ()))
    h1 = lax.dot_general(xg, w1, dn, preferred_element_type=jnp.float32)
    h2 = lax.dot_general(xg, w2, dn, preferred_element_type=jnp.float32)
    h = (h1 * jax.nn.sigmoid(h1)) * h2
    eo = lax.dot_general(h, wc, dn, preferred_element_type=jnp.float32)
    y_ref[...] = eo * sw_ref[:, 0:1]


def _expert_mlp(xg, w1, w2, wc, slotw, tile_eid):
    grid_spec = pltpu.PrefetchScalarGridSpec(
        num_scalar_prefetch=1,
        grid=(NTILES,),
        in_specs=[
            pl.BlockSpec((TILE, D), lambda i, eid: (i, 0)),
            pl.BlockSpec((1, H, D), lambda i, eid: (eid[i], 0, 0)),
            pl.BlockSpec((1, H, D), lambda i, eid: (eid[i], 0, 0)),
            pl.BlockSpec((1, D, H), lambda i, eid: (eid[i], 0, 0)),
            pl.BlockSpec((TILE, 128), lambda i, eid: (i, 0)),
        ],
        out_specs=pl.BlockSpec((TILE, D), lambda i, eid: (i, 0)),
    )
    return pl.pallas_call(
        _expert_body,
        grid_spec=grid_spec,
        out_shape=jax.ShapeDtypeStruct((NSLOTS, D), jnp.float32),
        compiler_params=pltpu.CompilerParams(
            dimension_semantics=("arbitrary",),
        ),
    )(tile_eid, xg, w1, w2, wc, slotw)


def kernel(x, W1, W2, Wc, Wg):
    b, s, d = x.shape
    x_flat = x.reshape(T, D)
    wg_pad = jnp.zeros((D, 128), jnp.float32).at[:, :NE].set(Wg.T)
    tri = (jnp.arange(128)[:, None] < jnp.arange(128)[None, :]
           ).astype(jnp.float32)
    ids, scores, cnts, offs, te = _router(x_flat, wg_pad, tri)
    pos, xg, slotw = _dispatch_kernel_build()(ids, cnts, offs, scores, x_flat)
    y = _expert_mlp(xg, W1, W2, Wc, slotw, te)
    out = _combine_kernel_build()(y, pos)
    return out.reshape(b, s, d)
